# baseline (speedup 1.0000x reference)
import functools

import jax
import jax.numpy as jnp
from jax import lax
from jax.experimental import pallas as pl
from jax.experimental.pallas import tpu as pltpu
from jax.experimental.pallas import tpu_sc as plsc

_K = 32
_BRE = 2048
_BHE = 512
_BRD = 2048
_BHD = 512

_NC = 2
_NS = 16
_NW = _NC * _NS
_L = 16

_I32_MIN = -2147483648


def _enc_kernel(x_ref, we_ref, be_ref, h_ref):
    h_ref[...] = jax.lax.dot_general(
        x_ref[...], we_ref[...], (((1,), (1,)), ((), ())),
        preferred_element_type=jnp.float32) + be_ref[...]


def _dec_kernel(h_ref, thr_ref, wd_ref, bd_ref, recon_ref, hs_ref):
    j = pl.program_id(1)
    h_tile = h_ref[...]
    hs = jnp.where(h_tile >= thr_ref[...], h_tile, jnp.float32(0.0))
    hs_ref[...] = hs
    part = jax.lax.dot_general(
        hs, wd_ref[...], (((1,), (1,)), ((), ())),
        preferred_element_type=jnp.float32)

    @pl.when(j == 0)
    def _():
        recon_ref[...] = part + bd_ref[...]

    @pl.when(j > 0)
    def _():
        recon_ref[...] += part


def _scal(v):
    return lax.reduce_max(v, (0,))


def _to_ukey(v):
    b = lax.bitcast_convert_type(v, jnp.int32)
    neg = lax.shift_right_arithmetic(b, 31)
    return b ^ (neg | jnp.int32(_I32_MIN))


def _find_bucket(hist_ref, nbins, kp):
    nv = nbins // _L
    iota = lax.iota(jnp.int32, _L)

    def body(t, state):
        carry, found, bstar, above = state
        vi = nv - 1 - t
        hv = hist_ref[pl.ds(vi * _L, _L)]
        rv = lax.rev(hv, (0,))
        tot = plsc.cumsum(rv) + carry
        m = tot >= kp
        pc = _scal(plsc.all_reduce_population_count(m))
        hit = jnp.logical_and(pc > 0, jnp.logical_not(found))
        j = plsc.all_reduce_ffs(m)
        binv = vi * _L + 15 - j
        totj = lax.reduce_sum(jnp.where(iota == j, tot, 0), (0,))
        rvj = lax.reduce_sum(jnp.where(iota == j, rv, 0), (0,))
        bstar = jnp.where(hit, _scal(binv), bstar)
        above = jnp.where(hit, totj - rvj, above)
        found = jnp.logical_or(found, pc > 0)
        carry = carry + lax.reduce_sum(hv, (0,))
        return (carry, found, bstar, above)

    init = (jnp.int32(0), jnp.bool_(False), jnp.int32(0), jnp.int32(0))
    _, _, bstar, above = lax.fori_loop(0, nv, body, init)
    return bstar, above


def _zero_hist(hist_ref):
    z = jnp.zeros((_L,), jnp.int32)
    for g in range((8 * 256) // _L):
        hist_ref[pl.ds(g * _L, _L)] = z


def _collapse_hist(hist_ref):
    for g in range(256 // _L):
        acc = hist_ref[pl.ds(g * _L, _L)]
        for u in range(1, 8):
            acc = acc + hist_ref[pl.ds(u * 256 + g * _L, _L)]
        hist_ref[pl.ds(g * _L, _L)] = acc


def _sc_thr_body(nrow, rows_pw, h_ref, thr_ref, buf0, buf1, keys,
                 hist, out_vec, sem0, sem1):
    wid = lax.axis_index("s") * _NC + lax.axis_index("c")
    base = wid * rows_pw
    nv = buf0.shape[0] // _L
    iota = lax.iota(jnp.int32, _L)
    ones = jnp.ones((_L,), jnp.int32)

    pltpu.async_copy(h_ref.at[base], buf0, sem0)
    pltpu.async_copy(h_ref.at[base + 1], buf1, sem1)

    def do_row(r, buf, sem):
        gr = base + r
        pltpu.make_async_copy(h_ref.at[gr], buf, sem).wait()

        gvr = nv // _K

        def grp(g, tlo):
            def inner(t, ms):
                m0, m1, m2, m3 = ms
                i = (g * gvr + t * 4) * _L
                m0 = jnp.maximum(m0, buf[pl.ds(i, _L)])
                m1 = jnp.maximum(m1, buf[pl.ds(i + _L, _L)])
                m2 = jnp.maximum(m2, buf[pl.ds(i + 2 * _L, _L)])
                m3 = jnp.maximum(m3, buf[pl.ds(i + 3 * _L, _L)])
                return (m0, m1, m2, m3)

            neg = jnp.full((_L,), -jnp.inf, jnp.float32)
            m0, m1, m2, m3 = lax.fori_loop(0, gvr // 4, inner,
                                           (neg, neg, neg, neg))
            gm = lax.reduce_max(jnp.maximum(jnp.maximum(m0, m1),
                                            jnp.maximum(m2, m3)), (0,))
            return jnp.minimum(tlo, gm)

        tlo = lax.fori_loop(0, _K, grp, jnp.float32(jnp.inf))
        ukl = _to_ukey(jnp.broadcast_to(tlo, (_L,)))

        _zero_hist(hist)

        def pass_a(t, c):
            for u in range(8):
                i = t * 8 + u
                v = buf[pl.ds(i * _L, _L)]
                uk = _to_ukey(v)
                keys[pl.ds(i * _L, _L)] = uk
                cnd = uk >= ukl
                d0 = lax.shift_right_logical(uk, 24) + (u * 256)
                plsc.addupdate_scatter(hist, [d0], ones, mask=cnd)
            return c

        lax.fori_loop(0, nv // 8, pass_a, jnp.int32(0))
        _collapse_hist(hist)
        b0, above0 = _find_bucket(hist, 256, _K)
        kp = _K - above0
        prefix = b0

        for lvl, sh in ((1, 16), (2, 8), (3, 0)):
            _zero_hist(hist)
            psh = sh + 8
            pfx = prefix

            def pass_f(t, c, psh=psh, sh=sh, pfx=pfx):
                for u in range(8):
                    i = t * 8 + u
                    uk = keys[pl.ds(i * _L, _L)]
                    match = jnp.logical_and(
                        uk >= ukl,
                        lax.shift_right_logical(uk, psh) == pfx)
                    d = (lax.shift_right_logical(uk, sh) & 0xFF) + (u * 256)
                    plsc.addupdate_scatter(hist, [d], ones, mask=match)
                return c

            lax.fori_loop(0, nv // 8, pass_f, jnp.int32(0))
            _collapse_hist(hist)
            b, above = _find_bucket(hist, 256, kp)
            kp = kp - above
            prefix = lax.shift_left(prefix, 8) | b

        uk = prefix
        bits = jnp.where(uk < 0, uk ^ jnp.int32(_I32_MIN), ~uk)
        thr = lax.bitcast_convert_type(bits, jnp.float32)
        lane = r & (_L - 1)
        out_vec[...] = jnp.where(iota == lane, thr, out_vec[...])

        @pl.when(lane == _L - 1)
        def _():
            off = pl.multiple_of(base + r - (_L - 1), _L)
            pltpu.sync_copy(out_vec, thr_ref.at[pl.ds(off, _L)])

        @pl.when(r + 2 < rows_pw)
        def _():
            pltpu.async_copy(h_ref.at[gr + 2], buf, sem)

    def pair(p, c):
        do_row(p * 2, buf0, sem0)
        do_row(p * 2 + 1, buf1, sem1)
        return c

    lax.fori_loop(0, rows_pw // 2, pair, jnp.int32(0))


def _sc_threshold(h):
    B, H = h.shape
    rows_pw = B // _NW
    mesh = plsc.VectorSubcoreMesh(core_axis_name="c", subcore_axis_name="s",
                                  num_cores=_NC, num_subcores=_NS)
    body = functools.partial(_sc_thr_body, B, rows_pw)
    return pl.kernel(
        body,
        out_type=jax.ShapeDtypeStruct((B,), jnp.float32),
        mesh=mesh,
        compiler_params=pltpu.CompilerParams(needs_layout_passes=False),
        scratch_types=[
            pltpu.VMEM((H,), jnp.float32),
            pltpu.VMEM((H,), jnp.float32),
            pltpu.VMEM((H,), jnp.int32),
            pltpu.VMEM((8 * 256,), jnp.int32),
            pltpu.VMEM((_L,), jnp.float32),
            pltpu.SemaphoreType.DMA,
            pltpu.SemaphoreType.DMA,
        ],
    )(h)


def kernel(x, W_e, b_e, W_d, b_d):
    B, D = x.shape
    H = W_e.shape[0]
    bre = _BRE if B % _BRE == 0 else B
    bhe = _BHE if H % _BHE == 0 else H
    brd = _BRD if B % _BRD == 0 else B
    bhd = _BHD if H % _BHD == 0 else H

    be2 = b_e.reshape(1, H)
    bd2 = b_d.reshape(1, D)

    h = pl.pallas_call(
        _enc_kernel,
        grid=(B // bre, H // bhe),
        in_specs=[
            pl.BlockSpec((bre, D), lambda i, j: (i, 0)),
            pl.BlockSpec((bhe, D), lambda i, j: (j, 0)),
            pl.BlockSpec((1, bhe), lambda i, j: (0, j)),
        ],
        out_specs=pl.BlockSpec((bre, bhe), lambda i, j: (i, j)),
        out_shape=jax.ShapeDtypeStruct((B, H), jnp.float32),
    )(x, W_e, be2)

    thr = _sc_threshold(h).reshape(B, 1)

    recon, hs = pl.pallas_call(
        _dec_kernel,
        grid=(B // brd, H // bhd),
        in_specs=[
            pl.BlockSpec((brd, bhd), lambda i, j: (i, j)),
            pl.BlockSpec((brd, 1), lambda i, j: (i, 0)),
            pl.BlockSpec((D, bhd), lambda i, j: (0, j)),
            pl.BlockSpec((1, D), lambda i, j: (0, 0)),
        ],
        out_specs=[
            pl.BlockSpec((brd, D), lambda i, j: (i, 0)),
            pl.BlockSpec((brd, bhd), lambda i, j: (i, j)),
        ],
        out_shape=[
            jax.ShapeDtypeStruct((B, D), jnp.float32),
            jax.ShapeDtypeStruct((B, H), jnp.float32),
        ],
    )(h, thr, W_d, bd2)
    return (recon, hs, h)


# speedup vs baseline: 3.3623x; 3.3623x over previous
import functools

import jax
import jax.numpy as jnp
from jax import lax
from jax.experimental import pallas as pl
from jax.experimental.pallas import tpu as pltpu
from jax.experimental.pallas import tpu_sc as plsc

_K = 32
_BRE = 2048
_BHE = 512
_BRD = 2048
_BHD = 512

_NC = 2
_NS = 16
_NW = _NC * _NS
_L = 16

_I32_MIN = -2147483648


def _enc_kernel(x_ref, we_ref, be_ref, h_ref):
    h_ref[...] = jax.lax.dot_general(
        x_ref[...], we_ref[...], (((1,), (1,)), ((), ())),
        preferred_element_type=jnp.float32) + be_ref[...]


def _dec_kernel(h_ref, thr_ref, wd_ref, bd_ref, recon_ref, hs_ref):
    j = pl.program_id(1)
    h_tile = h_ref[...]
    hs = jnp.where(h_tile >= thr_ref[...], h_tile, jnp.float32(0.0))
    hs_ref[...] = hs
    part = jax.lax.dot_general(
        hs, wd_ref[...], (((1,), (1,)), ((), ())),
        preferred_element_type=jnp.float32)

    @pl.when(j == 0)
    def _():
        recon_ref[...] = part + bd_ref[...]

    @pl.when(j > 0)
    def _():
        recon_ref[...] += part


def _scal(v):
    return lax.reduce_max(v, (0,))


def _to_ukey(v):
    b = lax.bitcast_convert_type(v, jnp.int32)
    neg = lax.shift_right_arithmetic(b, 31)
    return b ^ (neg | jnp.int32(_I32_MIN))


def _find_bucket(hist_ref, nbins, kp):
    nv = nbins // _L
    iota = lax.iota(jnp.int32, _L)

    def body(t, state):
        carry, found, bstar, above = state
        vi = nv - 1 - t
        hv = hist_ref[pl.ds(vi * _L, _L)]
        rv = lax.rev(hv, (0,))
        tot = plsc.cumsum(rv) + carry
        m = tot >= kp
        pc = _scal(plsc.all_reduce_population_count(m))
        hit = jnp.logical_and(pc > 0, jnp.logical_not(found))
        j = plsc.all_reduce_ffs(m)
        binv = vi * _L + 15 - j
        totj = lax.reduce_sum(jnp.where(iota == j, tot, 0), (0,))
        rvj = lax.reduce_sum(jnp.where(iota == j, rv, 0), (0,))
        bstar = jnp.where(hit, _scal(binv), bstar)
        above = jnp.where(hit, totj - rvj, above)
        found = jnp.logical_or(found, pc > 0)
        carry = carry + lax.reduce_sum(hv, (0,))
        return (carry, found, bstar, above)

    init = (jnp.int32(0), jnp.bool_(False), jnp.int32(0), jnp.int32(0))
    _, _, bstar, above = lax.fori_loop(0, nv, body, init)
    return bstar, above


def _zero_hist(hist_ref):
    z = jnp.zeros((_L,), jnp.int32)
    for g in range((8 * 256) // _L):
        hist_ref[pl.ds(g * _L, _L)] = z


def _collapse_hist(hist_ref):
    for g in range(256 // _L):
        acc = hist_ref[pl.ds(g * _L, _L)]
        for u in range(1, 8):
            acc = acc + hist_ref[pl.ds(u * 256 + g * _L, _L)]
        hist_ref[pl.ds(g * _L, _L)] = acc


def _sc_thr_body(nrow, rows_pw, h_ref, thr_ref, buf0, buf1, keys,
                 hist, out_vec, sem0, sem1):
    wid = lax.axis_index("s") * _NC + lax.axis_index("c")
    base = wid * rows_pw
    nv = buf0.shape[0] // _L
    iota = lax.iota(jnp.int32, _L)
    ones = jnp.ones((_L,), jnp.int32)

    pltpu.async_copy(h_ref.at[base], buf0, sem0)
    pltpu.async_copy(h_ref.at[base + 1], buf1, sem1)

    def do_row(r, buf, sem):
        gr = base + r
        pltpu.make_async_copy(h_ref.at[gr], buf, sem).wait()

        gvr = nv // _K

        def grp(g, tlo):
            neg = jnp.full((_L,), -jnp.inf, jnp.float32)

            @plsc.parallel_loop(0, gvr // 4, unroll=4,
                                carry=(neg, neg, neg, neg))
            def inner(t, ms):
                m0, m1, m2, m3 = ms
                i = (g * gvr + t * 4) * _L
                m0 = jnp.maximum(m0, buf[pl.ds(i, _L)])
                m1 = jnp.maximum(m1, buf[pl.ds(i + _L, _L)])
                m2 = jnp.maximum(m2, buf[pl.ds(i + 2 * _L, _L)])
                m3 = jnp.maximum(m3, buf[pl.ds(i + 3 * _L, _L)])
                return (m0, m1, m2, m3)

            m0, m1, m2, m3 = inner
            gm = lax.reduce_max(jnp.maximum(jnp.maximum(m0, m1),
                                            jnp.maximum(m2, m3)), (0,))
            return jnp.minimum(tlo, gm)

        tlo = lax.fori_loop(0, _K, grp, jnp.float32(jnp.inf))
        ukl = _to_ukey(jnp.broadcast_to(tlo, (_L,)))

        _zero_hist(hist)

        @plsc.parallel_loop(0, nv, unroll=8)
        def pass_a(i):
            v = buf[pl.ds(i * _L, _L)]
            uk = _to_ukey(v)
            keys[pl.ds(i * _L, _L)] = uk
            cnd = uk >= ukl
            d0 = lax.shift_right_logical(uk, 24) + ((i & 7) * 256)
            plsc.addupdate_scatter(hist, [d0], ones, mask=cnd)
        _collapse_hist(hist)
        b0, above0 = _find_bucket(hist, 256, _K)
        kp = _K - above0
        prefix = b0

        for lvl, sh in ((1, 16), (2, 8), (3, 0)):
            _zero_hist(hist)
            psh = sh + 8
            pfx = prefix

            @plsc.parallel_loop(0, nv, unroll=8)
            def pass_f(i, psh=psh, sh=sh, pfx=pfx):
                uk = keys[pl.ds(i * _L, _L)]
                match = jnp.logical_and(
                    uk >= ukl,
                    lax.shift_right_logical(uk, psh) == pfx)
                d = ((lax.shift_right_logical(uk, sh) & 0xFF)
                     + ((i & 7) * 256))
                plsc.addupdate_scatter(hist, [d], ones, mask=match)
            _collapse_hist(hist)
            b, above = _find_bucket(hist, 256, kp)
            kp = kp - above
            prefix = lax.shift_left(prefix, 8) | b

        uk = prefix
        bits = jnp.where(uk < 0, uk ^ jnp.int32(_I32_MIN), ~uk)
        thr = lax.bitcast_convert_type(bits, jnp.float32)
        lane = r & (_L - 1)
        out_vec[...] = jnp.where(iota == lane, thr, out_vec[...])

        @pl.when(lane == _L - 1)
        def _():
            off = pl.multiple_of(base + r - (_L - 1), _L)
            pltpu.sync_copy(out_vec, thr_ref.at[pl.ds(off, _L)])

        @pl.when(r + 2 < rows_pw)
        def _():
            pltpu.async_copy(h_ref.at[gr + 2], buf, sem)

    def pair(p, c):
        do_row(p * 2, buf0, sem0)
        do_row(p * 2 + 1, buf1, sem1)
        return c

    lax.fori_loop(0, rows_pw // 2, pair, jnp.int32(0))


def _sc_threshold(h):
    B, H = h.shape
    rows_pw = B // _NW
    mesh = plsc.VectorSubcoreMesh(core_axis_name="c", subcore_axis_name="s",
                                  num_cores=_NC, num_subcores=_NS)
    body = functools.partial(_sc_thr_body, B, rows_pw)
    return pl.kernel(
        body,
        out_type=jax.ShapeDtypeStruct((B,), jnp.float32),
        mesh=mesh,
        compiler_params=pltpu.CompilerParams(needs_layout_passes=False),
        scratch_types=[
            pltpu.VMEM((H,), jnp.float32),
            pltpu.VMEM((H,), jnp.float32),
            pltpu.VMEM((H,), jnp.int32),
            pltpu.VMEM((8 * 256,), jnp.int32),
            pltpu.VMEM((_L,), jnp.float32),
            pltpu.SemaphoreType.DMA,
            pltpu.SemaphoreType.DMA,
        ],
    )(h)


def kernel(x, W_e, b_e, W_d, b_d):
    B, D = x.shape
    H = W_e.shape[0]
    bre = _BRE if B % _BRE == 0 else B
    bhe = _BHE if H % _BHE == 0 else H
    brd = _BRD if B % _BRD == 0 else B
    bhd = _BHD if H % _BHD == 0 else H

    be2 = b_e.reshape(1, H)
    bd2 = b_d.reshape(1, D)

    h = pl.pallas_call(
        _enc_kernel,
        grid=(B // bre, H // bhe),
        in_specs=[
            pl.BlockSpec((bre, D), lambda i, j: (i, 0)),
            pl.BlockSpec((bhe, D), lambda i, j: (j, 0)),
            pl.BlockSpec((1, bhe), lambda i, j: (0, j)),
        ],
        out_specs=pl.BlockSpec((bre, bhe), lambda i, j: (i, j)),
        out_shape=jax.ShapeDtypeStruct((B, H), jnp.float32),
    )(x, W_e, be2)

    thr = _sc_threshold(h).reshape(B, 1)

    recon, hs = pl.pallas_call(
        _dec_kernel,
        grid=(B // brd, H // bhd),
        in_specs=[
            pl.BlockSpec((brd, bhd), lambda i, j: (i, j)),
            pl.BlockSpec((brd, 1), lambda i, j: (i, 0)),
            pl.BlockSpec((D, bhd), lambda i, j: (0, j)),
            pl.BlockSpec((1, D), lambda i, j: (0, 0)),
        ],
        out_specs=[
            pl.BlockSpec((brd, D), lambda i, j: (i, 0)),
            pl.BlockSpec((brd, bhd), lambda i, j: (i, j)),
        ],
        out_shape=[
            jax.ShapeDtypeStruct((B, D), jnp.float32),
            jax.ShapeDtypeStruct((B, H), jnp.float32),
        ],
    )(h, thr, W_d, bd2)
    return (recon, hs, h)
